# baseline (speedup 1.0000x reference)
import jax
import jax.numpy as jnp
from jax import lax
from jax.experimental import pallas as pl
from jax.experimental.pallas import tpu as pltpu
from jax.experimental.pallas import tpu_sc as plsc

N = 10000
E = 320000
D = 128
SLOPE = 0.2

NC = 2
NS = 16
NW = NC * NS
ET = E // NW
C = 80
NCHUNK = ET // C
NSLOT = 2
RPT = 624
TAIL = N - NS * RPT


def _tc_prep(x_ref, w_ref, b_ref, hc_ref, s_ref, t_ref, u1_ref):
    h = jnp.dot(x_ref[...], w_ref[...], preferred_element_type=jnp.float32)
    b = b_ref[...]
    s = lax.dot_general(h, b[:, :D], (((1,), (1,)), ((), ())))
    t = lax.dot_general(h, b[:, D:], (((1,), (1,)), ((), ())))
    s_ref[...] = s
    t_ref[...] = t
    u1_ref[...] = jnp.exp(s)
    hc_ref[:N] = jnp.exp(t) * h
    hc_ref[N:] = h


def _sc_edges(hc_hbm, s_hbm, t_hbm, src_hbm, trg_hbm,
              acc_hbm, den_hbm,
              s_tab, t_tab, srcb, trgb, idxb, rowsb, exb, wb, dbuf,
              gsem,
              acc_sh, den_sh):
    c = lax.axis_index("c")
    sid = lax.axis_index("s")
    wid = c * NS + sid
    ebase = wid * ET

    pltpu.sync_copy(s_hbm, s_tab)
    pltpu.sync_copy(t_hbm, t_tab)

    z16 = jnp.zeros((16,), jnp.float32)
    zbuf = rowsb[0]

    def zrow(i, carry):
        for j in range(8):
            zbuf[i, pl.ds(j * 16, 16)] = z16
        return carry

    lax.fori_loop(0, C, zrow, 0)

    def zden(i, carry):
        dbuf[pl.ds(i * 16, 16)] = z16
        return carry

    lax.fori_loop(0, 1000 // 16, zden, 0)

    for j in range(RPT // C):
        pltpu.sync_copy(zbuf, acc_sh.at[pl.ds(sid * RPT + j * C, C)])
    pltpu.sync_copy(zbuf.at[pl.ds(0, RPT - (RPT // C) * C)],
                    acc_sh.at[pl.ds(sid * RPT + (RPT // C) * C, RPT - (RPT // C) * C)])

    @pl.when(sid == 0)
    def _():
        pltpu.sync_copy(zbuf.at[pl.ds(0, TAIL)], acc_sh.at[pl.ds(NS * RPT, TAIL)])

    @pl.when(sid < 10)
    def _():
        pltpu.sync_copy(dbuf, den_sh.at[pl.ds(sid * 1000, 1000)])

    plsc.subcore_barrier()


    def prep_chunk(k, s):
        pltpu.sync_copy(src_hbm.at[pl.ds(ebase + k * C, C)], srcb[s])
        pltpu.sync_copy(trg_hbm.at[pl.ds(ebase + k * C, C)], trgb[s])
        src_v = srcb[s]
        trg_v = trgb[s]
        for g in range(C // 16):
            sl = pl.ds(g * 16, 16)
            si = src_v[sl]
            ti = trg_v[sl]
            sv = plsc.load_gather(s_tab, [si])
            tv = plsc.load_gather(t_tab, [ti])
            a = sv + tv
            neg = a < 0
            exb[s][sl] = jnp.exp(jnp.where(neg, a * SLOPE, a))
            wb[s][sl] = jnp.where(neg, jnp.exp(a * SLOPE - sv), 1.0)
            idxb[s][sl] = jnp.where(neg, ti + N, ti)
        pltpu.async_copy(hc_hbm.at[idxb[s]], rowsb[s], gsem[s])

    def drain_chunk(s):
        pltpu.make_async_copy(hc_hbm.at[idxb[s]], rowsb[s], gsem[s]).wait()
        rows = rowsb[s]
        w_v = wb[s]
        for g in range(C // 16):
            w_vec = w_v[pl.ds(g * 16, 16)]
            wdev = jnp.max(jnp.abs(w_vec - 1.0))

            @pl.when(wdev > 0.0)
            def _():
                for j in range(16):
                    r = g * 16 + j
                    bc = w_vec[jnp.full((16,), j, jnp.int32)]
                    for q in range(8):
                        sl = pl.ds(q * 16, 16)
                        rows[r, sl] = rows[r, sl] * bc

        pltpu.sync_copy(rows, acc_sh.at[srcb[s]], add=True)
        pltpu.sync_copy(exb[s], den_sh.at[srcb[s]], add=True)

    def step(k, s, prefetch_next):
        if prefetch_next:
            prep_chunk(k + 1, 1 - s)
        drain_chunk(s)

    prep_chunk(0, 0)
    step(0, 0, True)

    def pair2(i, carry):
        k = 1 + 2 * i
        step(k, 1, True)
        step(k + 1, 0, True)
        return carry

    lax.fori_loop(0, (NCHUNK - 3) // 2, pair2, 0)

    step(NCHUNK - 2, 1, True)
    step(NCHUNK - 1, 0, False)

    plsc.subcore_barrier()

    nfull = RPT // C
    rem = RPT - nfull * C
    for j in range(nfull):
        rb = sid * RPT + j * C
        bb = rowsb[0]
        pltpu.sync_copy(acc_sh.at[pl.ds(rb, C)], bb)
        pltpu.sync_copy(bb, acc_hbm.at[c, pl.ds(rb, C)])
    rb = sid * RPT + nfull * C
    pltpu.sync_copy(acc_sh.at[pl.ds(rb, rem)], rowsb[0].at[pl.ds(0, rem)])
    pltpu.sync_copy(rowsb[0].at[pl.ds(0, rem)], acc_hbm.at[c, pl.ds(rb, rem)])

    @pl.when(sid == 0)
    def _():
        pltpu.sync_copy(acc_sh.at[pl.ds(NS * RPT, TAIL)], rowsb[0].at[pl.ds(0, TAIL)])
        pltpu.sync_copy(rowsb[0].at[pl.ds(0, TAIL)], acc_hbm.at[c, pl.ds(NS * RPT, TAIL)])

    @pl.when(sid < 10)
    def _():
        pltpu.sync_copy(den_sh.at[pl.ds(sid * 1000, 1000)], dbuf)
        pltpu.sync_copy(dbuf, den_hbm.at[pl.ds(c * N + sid * 1000, 1000)])


def _tc_finish(acc_ref, den_ref, u1_ref, o_ref):
    a = acc_ref[0] + acc_ref[1]
    d = den_ref[0] + den_ref[1]
    o_ref[...] = u1_ref[...] * a / (d + 1e-16)


def kernel(input_matrix, adjacency_coo_matrix, weights_matrix, attention_bias_vector):
    hc, s, t, u1 = pl.pallas_call(
        _tc_prep,
        out_shape=[
            jax.ShapeDtypeStruct((2 * N, D), jnp.float32),
            jax.ShapeDtypeStruct((N, 1), jnp.float32),
            jax.ShapeDtypeStruct((N, 1), jnp.float32),
            jax.ShapeDtypeStruct((N, 1), jnp.float32),
        ],
    )(input_matrix, weights_matrix, attention_bias_vector)

    src = adjacency_coo_matrix[0]
    trg = adjacency_coo_matrix[1]

    mesh = plsc.VectorSubcoreMesh(core_axis_name="c", subcore_axis_name="s")
    acc, den = pl.kernel(
        _sc_edges,
        out_type=[
            jax.ShapeDtypeStruct((NC, N, D), jnp.float32),
            jax.ShapeDtypeStruct((NC * N,), jnp.float32),
        ],
        mesh=mesh,
        compiler_params=pltpu.CompilerParams(needs_layout_passes=False),
        scratch_types=[
            pltpu.VMEM((N,), jnp.float32),
            pltpu.VMEM((N,), jnp.float32),
            [pltpu.VMEM((C,), jnp.int32) for _ in range(NSLOT)],
            [pltpu.VMEM((C,), jnp.int32) for _ in range(NSLOT)],
            [pltpu.VMEM((C,), jnp.int32) for _ in range(NSLOT)],
            [pltpu.VMEM((C, D), jnp.float32) for _ in range(NSLOT)],
            [pltpu.VMEM((C,), jnp.float32) for _ in range(NSLOT)],
            [pltpu.VMEM((C,), jnp.float32) for _ in range(NSLOT)],
            pltpu.VMEM((1000,), jnp.float32),
            [pltpu.SemaphoreType.DMA for _ in range(NSLOT)],
            pltpu.VMEM_SHARED((N, D), jnp.float32),
            pltpu.VMEM_SHARED((N,), jnp.float32),
        ],
    )(hc, s.reshape(N), t.reshape(N), src, trg)

    out = pl.pallas_call(
        _tc_finish,
        out_shape=jax.ShapeDtypeStruct((N, D), jnp.float32),
    )(acc, den.reshape(NC, N, 1), u1)
    return out


# speedup vs baseline: 1.1036x; 1.1036x over previous
import jax
import jax.numpy as jnp
from jax import lax
from jax.experimental import pallas as pl
from jax.experimental.pallas import tpu as pltpu
from jax.experimental.pallas import tpu_sc as plsc

N = 10000
E = 320000
D = 128
SLOPE = 0.2

NC = 2
NS = 16
NW = NC * NS
ET = E // NW
C = 80
NCHUNK = ET // C
NSLOT = 2
RPT = 624
TAIL = N - NS * RPT


def _tc_prep(x_ref, w_ref, b_ref, h_ref, s_ref, t_ref):
    h = jnp.dot(x_ref[...], w_ref[...], preferred_element_type=jnp.float32)
    h_ref[...] = h
    b = b_ref[...]
    b1 = b[:, :D]
    b2 = b[:, D:]
    s_ref[...] = lax.dot_general(h, b1, (((1,), (1,)), ((), ())))
    t_ref[...] = lax.dot_general(h, b2, (((1,), (1,)), ((), ())))


def _sc_edges(h_hbm, s_hbm, t_hbm, src_hbm, trg_hbm,
              acc_hbm, den_hbm,
              s_tab, t_tab, srcb, trgb, rowsb, exb, dbuf,
              gsem, ssem, dsem,
              acc_sh, den_sh):
    c = lax.axis_index("c")
    sid = lax.axis_index("s")
    wid = c * NS + sid
    ebase = wid * ET

    pltpu.sync_copy(s_hbm, s_tab)
    pltpu.sync_copy(t_hbm, t_tab)

    z16 = jnp.zeros((16,), jnp.float32)
    zbuf = rowsb[0]

    def zrow(i, carry):
        for j in range(8):
            zbuf[i, pl.ds(j * 16, 16)] = z16
        return carry

    lax.fori_loop(0, C, zrow, 0)

    def zden(i, carry):
        dbuf[pl.ds(i * 16, 16)] = z16
        return carry

    lax.fori_loop(0, 1000 // 16, zden, 0)

    for j in range(RPT // C):
        pltpu.sync_copy(zbuf, acc_sh.at[pl.ds(sid * RPT + j * C, C)])
    pltpu.sync_copy(zbuf.at[pl.ds(0, RPT - (RPT // C) * C)],
                    acc_sh.at[pl.ds(sid * RPT + (RPT // C) * C, RPT - (RPT // C) * C)])

    @pl.when(sid == 0)
    def _():
        pltpu.sync_copy(zbuf.at[pl.ds(0, TAIL)], acc_sh.at[pl.ds(NS * RPT, TAIL)])

    @pl.when(sid < 10)
    def _():
        pltpu.sync_copy(dbuf, den_sh.at[pl.ds(sid * 1000, 1000)])

    plsc.subcore_barrier()


    def prep_chunk(k, s, first):
        if not first:
            pltpu.make_async_copy(rowsb[s], acc_sh.at[srcb[s]], ssem[s]).wait()
            pltpu.make_async_copy(exb[s], den_sh.at[srcb[s]], dsem[s]).wait()
        pltpu.sync_copy(src_hbm.at[pl.ds(ebase + k * C, C)], srcb[s])
        pltpu.sync_copy(trg_hbm.at[pl.ds(ebase + k * C, C)], trgb[s])
        pltpu.async_copy(h_hbm.at[trgb[s]], rowsb[s], gsem[s])

    def drain_chunk(s):
        pltpu.make_async_copy(h_hbm.at[trgb[s]], rowsb[s], gsem[s]).wait()
        rows = rowsb[s]
        ex_v = exb[s]
        src_v = srcb[s]
        trg_v = trgb[s]
        for g in range(C // 16):
            si = src_v[pl.ds(g * 16, 16)]
            ti = trg_v[pl.ds(g * 16, 16)]
            a = plsc.load_gather(s_tab, [si]) + plsc.load_gather(t_tab, [ti])
            a = jnp.where(a >= 0, a, a * SLOPE)
            ex_v[pl.ds(g * 16, 16)] = jnp.exp(a)

        for g in range(C // 16):
            ex_vec = ex_v[pl.ds(g * 16, 16)]
            for j in range(16):
                r = g * 16 + j
                bc = ex_vec[jnp.full((16,), j, jnp.int32)]
                for q in range(8):
                    sl = pl.ds(q * 16, 16)
                    rows[r, sl] = rows[r, sl] * bc

        pltpu.async_copy(rows, acc_sh.at[src_v], ssem[s], add=True)
        pltpu.async_copy(ex_v, den_sh.at[src_v], dsem[s], add=True)

    def step(k, s, prefetch_next, first=False):
        if prefetch_next:
            prep_chunk(k + 1, 1 - s, first)
        drain_chunk(s)

    prep_chunk(0, 0, True)
    step(0, 0, True, first=True)

    def pair2(i, carry):
        k = 1 + 2 * i
        step(k, 1, True)
        step(k + 1, 0, True)
        return carry

    lax.fori_loop(0, (NCHUNK - 3) // 2, pair2, 0)

    step(NCHUNK - 2, 1, True)
    step(NCHUNK - 1, 0, False)

    for s in range(NSLOT):
        pltpu.make_async_copy(rowsb[s], acc_sh.at[srcb[s]], ssem[s]).wait()
        pltpu.make_async_copy(exb[s], den_sh.at[srcb[s]], dsem[s]).wait()

    plsc.subcore_barrier()

    nfull = RPT // C
    rem = RPT - nfull * C
    for j in range(nfull):
        rb = sid * RPT + j * C
        bb = rowsb[0]
        pltpu.sync_copy(acc_sh.at[pl.ds(rb, C)], bb)
        pltpu.sync_copy(bb, acc_hbm.at[c, pl.ds(rb, C)])
    rb = sid * RPT + nfull * C
    pltpu.sync_copy(acc_sh.at[pl.ds(rb, rem)], rowsb[0].at[pl.ds(0, rem)])
    pltpu.sync_copy(rowsb[0].at[pl.ds(0, rem)], acc_hbm.at[c, pl.ds(rb, rem)])

    @pl.when(sid == 0)
    def _():
        pltpu.sync_copy(acc_sh.at[pl.ds(NS * RPT, TAIL)], rowsb[0].at[pl.ds(0, TAIL)])
        pltpu.sync_copy(rowsb[0].at[pl.ds(0, TAIL)], acc_hbm.at[c, pl.ds(NS * RPT, TAIL)])

    @pl.when(sid < 10)
    def _():
        pltpu.sync_copy(den_sh.at[pl.ds(sid * 1000, 1000)], dbuf)
        pltpu.sync_copy(dbuf, den_hbm.at[pl.ds(c * N + sid * 1000, 1000)])


def _tc_finish(acc_ref, den_ref, o_ref):
    a = acc_ref[0] + acc_ref[1]
    d = den_ref[0] + den_ref[1]
    o_ref[...] = a / (d + 1e-16)


def kernel(input_matrix, adjacency_coo_matrix, weights_matrix, attention_bias_vector):
    h, s, t = pl.pallas_call(
        _tc_prep,
        out_shape=[
            jax.ShapeDtypeStruct((N, D), jnp.float32),
            jax.ShapeDtypeStruct((N, 1), jnp.float32),
            jax.ShapeDtypeStruct((N, 1), jnp.float32),
        ],
    )(input_matrix, weights_matrix, attention_bias_vector)

    src = adjacency_coo_matrix[0]
    trg = adjacency_coo_matrix[1]

    mesh = plsc.VectorSubcoreMesh(core_axis_name="c", subcore_axis_name="s")
    acc, den = pl.kernel(
        _sc_edges,
        out_type=[
            jax.ShapeDtypeStruct((NC, N, D), jnp.float32),
            jax.ShapeDtypeStruct((NC * N,), jnp.float32),
        ],
        mesh=mesh,
        compiler_params=pltpu.CompilerParams(needs_layout_passes=False),
        scratch_types=[
            pltpu.VMEM((N,), jnp.float32),
            pltpu.VMEM((N,), jnp.float32),
            [pltpu.VMEM((C,), jnp.int32) for _ in range(NSLOT)],
            [pltpu.VMEM((C,), jnp.int32) for _ in range(NSLOT)],
            [pltpu.VMEM((C, D), jnp.float32) for _ in range(NSLOT)],
            [pltpu.VMEM((C,), jnp.float32) for _ in range(NSLOT)],
            pltpu.VMEM((1000,), jnp.float32),
            [pltpu.SemaphoreType.DMA for _ in range(NSLOT)],
            [pltpu.SemaphoreType.DMA for _ in range(NSLOT)],
            [pltpu.SemaphoreType.DMA for _ in range(NSLOT)],
            pltpu.VMEM_SHARED((N, D), jnp.float32),
            pltpu.VMEM_SHARED((N,), jnp.float32),
        ],
    )(h, s.reshape(N), t.reshape(N), src, trg)

    out = pl.pallas_call(
        _tc_finish,
        out_shape=jax.ShapeDtypeStruct((N, D), jnp.float32),
    )(acc, den.reshape(NC, N, 1))
    return out
